# baseline (speedup 1.0000x reference)
import functools

import jax
import jax.numpy as jnp
from jax import lax
from jax.experimental import pallas as pl
from jax.experimental.pallas import tpu as pltpu
from jax.experimental.pallas import tpu_sc as plsc

_T = 100
_C = 1000
_B = 1024
_L = 16
_NFULL = 62
_TAIL = 984
_NW = 32
_EPW = _B // _NW
_TPS = 7

_MESH = plsc.VectorSubcoreMesh(core_axis_name="c", subcore_axis_name="s")


def _xlane(v, op):
    i = lax.iota(jnp.int32, _L)
    for sh in (8, 4, 2, 1):
        p = jnp.bitwise_xor(i, sh)
        v = op(v, v.at[p].get(mode="promise_in_bounds"))
    return v


@functools.partial(
    pl.kernel,
    mesh=_MESH,
    out_type=jax.ShapeDtypeStruct((_NW, _L), jnp.float32),
    scratch_types=[
        pltpu.VMEM_SHARED((_T, _C), jnp.float32),
        pltpu.VMEM((_EPW, _C), jnp.float32),
        pltpu.VMEM((_EPW,), jnp.int32),
        pltpu.VMEM((_L,), jnp.int32),
        pltpu.VMEM((_L, _C), jnp.float32),
        pltpu.VMEM((_TPS * _C,), jnp.float32),
        pltpu.VMEM((_TPS * _L,), jnp.float32),
        pltpu.VMEM((_C,), jnp.float32),
        pltpu.VMEM((_L,), jnp.float32),
        pltpu.SemaphoreType.DMA,
        pltpu.SemaphoreType.DMA,
        pltpu.SemaphoreType.DMA,
        pltpu.SemaphoreType.DMA,
    ],
    compiler_params=pltpu.CompilerParams(use_tc_tiling_on_sc=False),
)
def _sc_weighted_loss(losses_hbm, idx_hbm, params_hbm, out_hbm,
                      acc, loss_v, idx_v, tidx_v, prows_v, e_v, r_v,
                      zrow_v, part_v, sem_l, sem_p, sem_z, sem_s):
    cid = lax.axis_index("c")
    sid = lax.axis_index("s")
    wid = cid * 16 + sid
    base = wid * (_EPW * _C)
    zvec = jnp.zeros((_L,), jnp.float32)
    lane = lax.iota(jnp.int32, _L)
    himask = lane >= 8

    with jax.named_scope("ph_stage_fire"):
        stages = [
            pltpu.async_copy(losses_hbm.at[pl.ds(base + e * _C, _C)],
                             loss_v.at[e], sem_l)
            for e in range(_EPW)
        ]
        pltpu.sync_copy(idx_hbm.at[pl.ds(wid * _EPW, _EPW)], idx_v)

    with jax.named_scope("ph_pgather"):
        tidx_v[...] = jnp.minimum(sid + 16 * lane, _T - 1)
        pgather = pltpu.async_copy(params_hbm.at[tidx_v], prows_v, sem_p)

    with jax.named_scope("ph_zero"):
        for j in range(_C // _L):
            zrow_v[pl.ds(j * _L, _L)] = zvec
        zrow_v[pl.ds(_C - _L, _L)] = zvec
        zcopies = [
            pltpu.async_copy(
                zrow_v, acc.at[jnp.minimum(sid + 16 * k, _T - 1)], sem_z)
            for k in range(_TPS)
        ]
        for cp in zcopies:
            cp.wait()

    with jax.named_scope("ph_bar1"):
        plsc.subcore_barrier()

    with jax.named_scope("ph_scatter_fire"):
        for cp in stages:
            cp.wait()
        scat = pltpu.async_copy(loss_v, acc.at[idx_v], sem_s, add=True)

    ntab = jnp.where(sid < _T - 16 * (_TPS - 1), _TPS, _TPS - 1)

    with jax.named_scope("ph_A_softmax"):
        pgather.wait()

        def _ta(k, carry):
            def _mb(j, m):
                return jnp.maximum(m, prows_v[k, pl.ds(j * _L, _L)])
            mvec = lax.fori_loop(0, _NFULL, _mb,
                                 prows_v[k, pl.ds(_TAIL, _L)], unroll=8)
            m = _xlane(mvec, jnp.maximum)

            def _eb(j, s):
                e = jnp.exp(prows_v[k, pl.ds(j * _L, _L)] - m)
                e_v[pl.ds(k * _C + j * _L, _L)] = e
                return s + e
            svec = lax.fori_loop(0, _NFULL, _eb, zvec, unroll=8)
            et = jnp.exp(prows_v[k, pl.ds(_TAIL, _L)] - m)
            e_v[pl.ds(k * _C + _TAIL, _L)] = et
            svec = svec + jnp.where(himask, et, 0.0)
            r_v[pl.ds(k * _L, _L)] = 1.0 / _xlane(svec, jnp.add)
            return carry

        lax.fori_loop(0, ntab, _ta, 0)

    with jax.named_scope("ph_scat_wait"):
        scat.wait()
    with jax.named_scope("ph_bar2"):
        plsc.subcore_barrier()

    with jax.named_scope("ph_B_dot"):
        pltpu.async_copy(acc.at[tidx_v], prows_v, sem_p).wait()

        def _tb(k, pv):
            def _db(j, a):
                return a + (prows_v[k, pl.ds(j * _L, _L)]
                            * e_v[pl.ds(k * _C + j * _L, _L)])
            part = lax.fori_loop(0, _NFULL, _db, zvec, unroll=8)
            pt = (prows_v[k, pl.ds(_TAIL, _L)]
                  * e_v[pl.ds(k * _C + _TAIL, _L)])
            part = part + jnp.where(himask, pt, 0.0)
            return pv + part * r_v[pl.ds(k * _L, _L)]

        pv = lax.fori_loop(0, ntab, _tb, zvec)

    with jax.named_scope("ph_out"):
        part_v[...] = pv
        pltpu.sync_copy(part_v, out_hbm.at[wid])


def kernel(losses, inputs_idx, params):
    partials = _sc_weighted_loss(losses, inputs_idx, params)
    return jnp.sum(partials)
